# baseline (speedup 1.0000x reference)
import functools
import math

import jax
import jax.numpy as jnp
from jax import lax
from jax.experimental import pallas as pl
from jax.experimental.pallas import tpu as pltpu
from jax.experimental.pallas import tpu_sc as plsc

N = 4096
NE = 4096
E = 262144
D = 128
H = 4
DH = D // H

NC = 2
NS = 16
NW = NC * NS
LANES = 16

CH = 64
IDX_ROWS = E // CH
ROWS_PER_W = IDX_ROWS // NW
GROUPS = ROWS_PER_W // 2
SLICE = 4096 // NS


CW = 128


def _seg_body(table, gi, si, out, cnt_out, gi_v, si_v, rows_v, acc, sem):
    c = lax.axis_index("c")
    s = lax.axis_index("s")
    wid = s * NC + c
    base = wid * ROWS_PER_W
    row0 = s * SLICE

    zeros16 = jnp.zeros((16,), jnp.float32)
    ones16 = jnp.ones((16,), jnp.float32)
    for r in range(CH):
        for k in range(D // 16):
            rows_v[0, r, pl.ds(k * 16, 16)] = zeros16

    for off in range(0, SLICE, CH):
        pltpu.sync_copy(rows_v.at[0], acc.at[pl.ds(row0 + off, CH)])
    plsc.subcore_barrier()

    def step(g, carry):
        r = base + g * 2
        pltpu.sync_copy(gi.at[pl.ds(r, 2)], gi_v)
        pltpu.sync_copy(si.at[pl.ds(r, 2)], si_v)
        d0 = pltpu.async_copy(table.at[gi_v.at[0]], rows_v.at[0], sem)
        d1 = pltpu.async_copy(table.at[gi_v.at[1]], rows_v.at[1], sem)
        for j, d in ((0, d0), (1, d1)):
            d.wait()
            pltpu.sync_copy(rows_v.at[j], acc.at[si_v.at[j]], add=True)
        return carry

    lax.fori_loop(0, GROUPS, step, 0)
    plsc.subcore_barrier()
    pltpu.sync_copy(acc.at[pl.ds(row0, SLICE)], out.at[c, pl.ds(row0, SLICE)])

    for r in range(CH):
        for k in range(D // 16):
            rows_v[0, r, pl.ds(k * 16, 16)] = zeros16
            rows_v[1, r, pl.ds(k * 16, 16)] = ones16
    for off in range(0, SLICE, CH):
        pltpu.sync_copy(rows_v.at[0], acc.at[pl.ds(row0 + off, CH)])
    plsc.subcore_barrier()

    def cstep(g, carry):
        r = base + g * 2
        pltpu.sync_copy(si.at[pl.ds(r, 2)], si_v)
        for j in range(2):
            pltpu.sync_copy(rows_v.at[1], acc.at[si_v.at[j]], add=True)
        return carry

    lax.fori_loop(0, GROUPS, cstep, 0)
    plsc.subcore_barrier()
    pltpu.sync_copy(acc.at[pl.ds(row0, SLICE)],
                    cnt_out.at[c, pl.ds(row0, SLICE)])


def _sc_mesh():
    return plsc.VectorSubcoreMesh(core_axis_name="c", subcore_axis_name="s",
                                  num_cores=NC, num_subcores=NS)


def _make_seg_call():
    outs = [jax.ShapeDtypeStruct((NC, 4096, D), jnp.float32),
            jax.ShapeDtypeStruct((NC, 4096, CW), jnp.float32)]
    scratch = [
        pltpu.VMEM((2, CH), jnp.int32),
        pltpu.VMEM((2, CH), jnp.int32),
        pltpu.VMEM((2, CH, D), jnp.float32),
        pltpu.VMEM_SHARED((4096, D), jnp.float32),
        pltpu.SemaphoreType.DMA,
    ]
    return pl.kernel(_seg_body, out_type=tuple(outs), mesh=_sc_mesh(),
                     scratch_types=tuple(scratch))


def _dot(a, b):
    return jnp.dot(a.astype(jnp.bfloat16), b.astype(jnp.bfloat16),
                   preferred_element_type=jnp.float32)


def _dot4(a, b):
    ahi = a.astype(jnp.bfloat16).astype(jnp.float32)
    alo = a - ahi
    bhi = b.astype(jnp.bfloat16).astype(jnp.float32)
    blo = b - bhi
    return (_dot(alo, blo) + (_dot(ahi, blo) + _dot(alo, bhi))) + _dot(ahi, bhi)


def _lstm(z, wih, bih, bhh):
    gates = _dot(z, wih.T) + bih + bhh
    i = jax.nn.sigmoid(gates[:, :D])
    g = jnp.tanh(gates[:, 2 * D:3 * D])
    o = jax.nn.sigmoid(gates[:, 3 * D:])
    return o * jnp.tanh(i * g)


def _pre_body(i1, whg, w4, b4, wih, bih, bhh, xw_o, fea2_o):
    x1 = i1[...]
    xw_o[...] = _dot(x1, whg[...].T)
    z = _dot(x1, w4[...].T) + b4[...]
    fea2_o[...] = _lstm(z, wih[...], bih[...], bhh[...])


def _henorm_body(hp, dep, out):
    h = hp[...]
    d = dep[...]
    de = d[0, :, 0:1] + d[1, :, 0:1]
    be = jnp.where(de > 0, 1.0 / de, 0.0)
    out[...] = (h[0] + h[1]) * be


QB = 256


def _att_body(xp, dvp, bhg, fea2, wih, bih, bhh, wq, bq, wk, bk, wv, bv,
              wo, bo, wc1, bc1, wc2, bc2, out, k_s, v_s):
    pi = pl.program_id(0)

    @pl.when(pi == 0)
    def _init():
        d = dvp[...]
        dv = d[0, :, 0:1] + d[1, :, 0:1]
        dvinv = jnp.where(dv > 0, 1.0 / dv, 0.0)
        xs = xp[...]
        x = (xs[0] + xs[1]) * dvinv + bhg[...]
        x = _lstm(x, wih[...], bih[...], bhh[...])
        k_s[...] = _dot(x, wk[...].T) + bk[...]
        v_s[...] = _dot(fea2[...], wv[...].T) + bv[...]

    f2b = fea2[pl.ds(pi * QB, QB), :]
    Qb = _dot(f2b, wq[...].T) + bq[...]
    K = k_s[...]
    V = v_s[...]
    ctxs = []
    for h in range(H):
        Qh = Qb[:, h * DH:(h + 1) * DH]
        Kh = K[:, h * DH:(h + 1) * DH]
        Vh = V[:, h * DH:(h + 1) * DH]
        S = _dot(Qh, Kh.T) / jnp.sqrt(jnp.float32(DH))
        m = jnp.max(S, axis=-1, keepdims=True)
        P = jnp.exp(S - m)
        A = P / jnp.sum(P, axis=-1, keepdims=True)
        ctxs.append(_dot(A, Vh))
    ctxb = jnp.concatenate(ctxs, axis=1)
    sh = _dot(ctxb, wo[...].T) + bo[...]
    h1 = jnp.maximum(_dot(sh, wc1[...].T) + bc1[...], 0.0)
    out[...] = _dot(h1, wc2[...].T) + bc2[...]


def _att_call(x_p, dv_p, b_hg, fea2, W_ih, b_ih, b_hh, Wq, bq, Wk, bk,
              Wv, bv, Wo, bo, Wc1, bc1, Wc2, bc2, interpret=False):
    full = lambda *shape: pl.BlockSpec(shape, lambda i: (0,) * len(shape))
    return pl.pallas_call(
        _att_body,
        grid=(N // QB,),
        in_specs=[
            full(NC, N, D),
            full(NC, N, CW),
            full(D),
            full(N, D),
            full(4 * D, D), full(4 * D), full(4 * D),
            full(D, D), full(D), full(D, D), full(D),
            full(D, D), full(D), full(D, D), full(D),
            full(D // 2, D), full(D // 2),
            full(2, D // 2), full(2),
        ],
        out_specs=pl.BlockSpec((QB, 2), lambda i: (i, 0)),
        out_shape=jax.ShapeDtypeStruct((N, 2), jnp.float32),
        scratch_shapes=[pltpu.VMEM((N, D), jnp.float32),
                        pltpu.VMEM((N, D), jnp.float32)],
        interpret=interpret,
    )(x_p, dv_p, b_hg, fea2, W_ih, b_ih, b_hh, Wq, bq, Wk, bk, Wv, bv,
      Wo, bo, Wc1, bc1, Wc2, bc2)


def kernel(fea, edge_index, edge_attr, edge_weights, l, W_hg, b_hg, W4, b4,
           W_ih, W_hh, b_ih, b_hh, Wq, bq, Wk, bk, Wv, bv, Wo, bo,
           Wc1, bc1, Wc2, bc2):
    input1 = fea[:, 1:]
    ni = edge_index[0].reshape(IDX_ROWS, CH)
    hi = edge_index[1].reshape(IDX_ROWS, CH)

    xw, fea2 = pl.pallas_call(
        _pre_body,
        out_shape=(jax.ShapeDtypeStruct((N, D), jnp.float32),
                   jax.ShapeDtypeStruct((N, D), jnp.float32)),
    )(input1, W_hg, W4, b4, W_ih, b_ih, b_hh)

    he_p, de_p = _make_seg_call()(xw, ni, hi)

    he_norm = pl.pallas_call(
        _henorm_body,
        out_shape=jax.ShapeDtypeStruct((NE, D), jnp.float32),
    )(he_p, de_p)

    x_p, dv_p = _make_seg_call()(he_norm, hi, ni)

    out = _att_call(x_p, dv_p, b_hg, fea2, W_ih, b_ih, b_hh, Wq, bq, Wk, bk,
                    Wv, bv, Wo, bo, Wc1, bc1, Wc2, bc2)
    return out
